# Initial kernel scaffold; baseline (speedup 1.0000x reference)
import jax
import jax.numpy as jnp
from jax.experimental import pallas as pl


def kernel(x, edge_index, edge_attr, pos, batch, W, b, attn, Wr, br):
    raise NotImplementedError("write your pallas kernel here")


# speedup vs baseline: 1.0617x; 1.0617x over previous
import jax
import jax.numpy as jnp
from jax.experimental import pallas as pl
from jax.experimental.pallas import tpu as pltpu


def _matmul_body(x_ref, w_ref, b_ref, o_ref):
    o_ref[...] = (
        jnp.dot(x_ref[...], w_ref[...].T, preferred_element_type=jnp.float32)
        + b_ref[...]
    )


def _linear(x, W, b):
    N, D = x.shape
    return pl.pallas_call(
        _matmul_body,
        out_shape=jax.ShapeDtypeStruct((N, D), jnp.float32),
    )(x, W, b[None, :])


def kernel(x, edge_index, edge_attr, pos, batch, W, b, attn, Wr, br):
    num_nodes = x.shape[0]
    G = 16
    a = jnp.minimum(edge_index[0], edge_index[1])
    bb = jnp.maximum(edge_index[0], edge_index[1])
    ids = a * num_nodes + bb
    ids_sorted = jnp.sort(ids)
    keep = jnp.concatenate(
        [jnp.ones((1,), dtype=bool), ids_sorted[1:] != ids_sorted[:-1]])
    a_s = (ids_sorted // num_nodes).astype(jnp.int32)
    b_s = (ids_sorted % num_nodes).astype(jnp.int32)
    present = jnp.zeros((num_nodes,), dtype=bool).at[edge_index.ravel()].set(True)

    all_nodes = jnp.arange(num_nodes, dtype=jnp.int32)
    src = jnp.concatenate([a_s, all_nodes])
    dst = jnp.concatenate([b_s, all_nodes])
    valid = jnp.concatenate([keep, ~present])

    h = _linear(x, W, b)
    x_src = jnp.take(h, src, axis=0)
    x_dst = jnp.take(h, dst, axis=0)
    h2 = jax.nn.elu(x_src + x_dst)
    atom_repr = jax.nn.elu(h2 * attn)
    batch_e = jnp.take(batch, src, axis=0)
    weighted = atom_repr @ Wr.T + br
    score = jax.nn.sigmoid(weighted)
    contrib = jnp.where(valid[:, None], score * atom_repr, 0.0)
    out1 = jax.ops.segment_sum(contrib, batch_e, num_segments=G)
    max_vals = jnp.where(valid[:, None], atom_repr, -jnp.inf)
    out2 = jax.ops.segment_max(max_vals, batch_e, num_segments=G)
    return jnp.concatenate([out1, out2], axis=1)
